# baseline (speedup 1.0000x reference)
import functools

import jax
import jax.numpy as jnp
from jax import lax
from jax.experimental import pallas as pl
from jax.experimental.pallas import tpu as pltpu
from jax.experimental.pallas import tpu_sc as plsc

_THRESHOLD = 0.4

_INFO = plsc.get_sparse_core_info()
_NC = _INFO.num_cores
_NS = _INFO.num_subcores
_NW = _NC * _NS
_L = _INFO.num_lanes


@functools.partial(jax.jit, static_argnames=("B", "K", "H", "W"))
def _run(c_flat, hm_flat, B, K, H, W):
    groups = B // _L

    mesh = plsc.VectorSubcoreMesh(
        core_axis_name="c", subcore_axis_name="s", num_cores=1
    )

    def body(c_hbm, hm_hbm, out_hbm, c_v, idx_v, vals_v, out_v, sem):
        wid = lax.axis_index("s")

        def do_k(k, base, n_groups):
            pltpu.sync_copy(
                c_hbm.at[pl.ds(k * 2 * B + base, n_groups * _L)],
                c_v.at[pl.ds(0, n_groups * _L)],
            )
            pltpu.sync_copy(
                c_hbm.at[pl.ds(k * 2 * B + B + base, n_groups * _L)],
                c_v.at[pl.ds(B, n_groups * _L)],
            )
            valids = []
            for g in range(n_groups):
                uu = c_v[pl.ds(g * _L, _L)]
                vv = c_v[pl.ds(B + g * _L, _L)]
                valid = (uu > -1) & (vv > -1) & (uu < W) & (vv < H)
                uc = jnp.clip(uu, 0, W - 1)
                vc = jnp.clip(vv, 0, H - 1)
                b = base + g * _L + lax.iota(jnp.int32, _L)
                idx_v[pl.ds(g * _L, _L)] = ((k * H + vc) * W + uc) * B + b
                valids.append(valid)
            if n_groups == groups:
                pltpu.async_copy(hm_hbm.at[idx_v], vals_v, sem).wait()
            else:
                pltpu.async_copy(
                    hm_hbm.at[idx_v.at[pl.ds(0, n_groups * _L)]],
                    vals_v.at[pl.ds(0, n_groups * _L)],
                    sem,
                ).wait()
            one = jnp.full((_L,), 1, jnp.int32)
            zero = jnp.full((_L,), 0, jnp.int32)
            for g in range(n_groups):
                vals = vals_v[pl.ds(g * _L, _L)]
                hit = (vals > _THRESHOLD) & valids[g]
                out_v[pl.ds(g * _L, _L)] = jnp.where(hit, one, zero)
            pltpu.sync_copy(
                out_v.at[pl.ds(0, n_groups * _L)],
                out_hbm.at[pl.ds(k * B + base, n_groups * _L)],
            )

        do_k(wid, 0, groups)

        @pl.when(wid < 2 * (K - _NS))
        def _():
            do_k(_NS, wid * (B // 2), groups // 2)

    run = pl.kernel(
        body,
        out_type=jax.ShapeDtypeStruct((K * B,), jnp.int32),
        mesh=mesh,
        compiler_params=pltpu.CompilerParams(
            needs_layout_passes=False,
            disable_bounds_checks=True,
            skip_device_barrier=True,
        ),
        scratch_types=[
            pltpu.VMEM((2 * B,), jnp.int32),
            pltpu.VMEM((B,), jnp.int32),
            pltpu.VMEM((B,), jnp.float32),
            pltpu.VMEM((B,), jnp.int32),
            pltpu.SemaphoreType.DMA,
        ],
    )
    out = run(c_flat, hm_flat)
    return (out > 0).reshape(K, B).T


def kernel(coords, heatmaps):
    B, K, H, W = heatmaps.shape
    c_flat = coords.astype(jnp.int32).transpose(1, 2, 0).reshape(-1)
    hm_flat = heatmaps.transpose(1, 2, 3, 0).reshape(-1)
    return _run(c_flat, hm_flat, B, K, H, W)


# speedup vs baseline: 1.0343x; 1.0343x over previous
import functools

import jax
import jax.numpy as jnp
from jax import lax
from jax.experimental import pallas as pl
from jax.experimental.pallas import tpu as pltpu
from jax.experimental.pallas import tpu_sc as plsc

_THRESHOLD = 0.4

_INFO = plsc.get_sparse_core_info()
_NC = _INFO.num_cores
_NS = _INFO.num_subcores
_NW = _NC * _NS
_L = _INFO.num_lanes


@functools.partial(jax.jit, static_argnames=("B", "K", "H", "W"))
def _run(c_flat, hm_flat, B, K, H, W):
    groups = B // _L

    mesh = plsc.VectorSubcoreMesh(
        core_axis_name="c", subcore_axis_name="s", num_cores=1
    )

    def body(c_hbm, hm_hbm, out_hbm, c_v, idx_v, vals_v, out_v, sem):
        wid = lax.axis_index("s")

        def do_k(k):
            pltpu.sync_copy(c_hbm.at[pl.ds(k * 2 * B, 2 * B)], c_v)
            valids = []
            for g in range(groups):
                uu = c_v[pl.ds(g * _L, _L)]
                vv = c_v[pl.ds(B + g * _L, _L)]
                valid = (uu > -1) & (vv > -1) & (uu < W) & (vv < H)
                uc = jnp.clip(uu, 0, W - 1)
                vc = jnp.clip(vv, 0, H - 1)
                b = g * _L + lax.iota(jnp.int32, _L)
                idx_v[pl.ds(g * _L, _L)] = ((k * H + vc) * W + uc) * B + b
                valids.append(valid)
            pltpu.async_copy(hm_hbm.at[idx_v], vals_v, sem).wait()
            one = jnp.full((_L,), 1, jnp.int32)
            zero = jnp.full((_L,), 0, jnp.int32)
            for g in range(groups):
                vals = vals_v[pl.ds(g * _L, _L)]
                hit = (vals > _THRESHOLD) & valids[g]
                out_v[pl.ds(g * _L, _L)] = jnp.where(hit, one, zero)
            pltpu.sync_copy(out_v, out_hbm.at[pl.ds(k * B, B)])

        do_k(wid)

        @pl.when(wid < K - _NS)
        def _():
            do_k(wid + _NS)

    run = pl.kernel(
        body,
        out_type=jax.ShapeDtypeStruct((K * B,), jnp.int32),
        mesh=mesh,
        compiler_params=pltpu.CompilerParams(
            needs_layout_passes=False,
            disable_bounds_checks=True,
            skip_device_barrier=True,
        ),
        scratch_types=[
            pltpu.VMEM((2 * B,), jnp.int32),
            pltpu.VMEM((B,), jnp.int32),
            pltpu.VMEM((B,), jnp.float32),
            pltpu.VMEM((B,), jnp.int32),
            pltpu.SemaphoreType.DMA,
        ],
    )
    out = run(c_flat, hm_flat)
    return (out > 0).reshape(K, B).T


def kernel(coords, heatmaps):
    B, K, H, W = heatmaps.shape
    c_flat = coords.astype(jnp.int32).transpose(1, 2, 0).reshape(-1)
    hm_flat = heatmaps.transpose(1, 2, 3, 0).reshape(-1)
    return _run(c_flat, hm_flat, B, K, H, W)


# speedup vs baseline: 1.0400x; 1.0055x over previous
import functools

import jax
import jax.numpy as jnp
from jax import lax
from jax.experimental import pallas as pl
from jax.experimental.pallas import tpu as pltpu
from jax.experimental.pallas import tpu_sc as plsc

_THRESHOLD = 0.4

_INFO = plsc.get_sparse_core_info()
_NC = _INFO.num_cores
_NS = _INFO.num_subcores
_NW = _NC * _NS
_L = _INFO.num_lanes


@functools.partial(jax.jit, static_argnames=("B", "K", "H", "W"))
def _run(c_flat, hm_flat, B, K, H, W):
    groups = B // _L

    mesh = plsc.VectorSubcoreMesh(
        core_axis_name="c", subcore_axis_name="s", num_cores=1
    )

    def body(c_hbm, hm_hbm, out_hbm, c_v, idx_v, vals_v, out_v, sem):
        wid = lax.axis_index("s")

        def do_k(k):
            pltpu.sync_copy(c_hbm.at[pl.ds(k * 2 * B, 2 * B)], c_v)
            valids = []
            for g in range(groups):
                uu = c_v[pl.ds(g * _L, _L)]
                vv = c_v[pl.ds(B + g * _L, _L)]
                valid = (uu > -1) & (vv > -1) & (uu < W) & (vv < H)
                uc = jnp.clip(uu, 0, W - 1)
                vc = jnp.clip(vv, 0, H - 1)
                b = g * _L + lax.iota(jnp.int32, _L)
                idx_v[pl.ds(g * _L, _L)] = ((k * H + vc) * W + uc) * B + b
                valids.append(valid)
            pltpu.async_copy(hm_hbm.at[idx_v], vals_v, sem).wait()
            one = jnp.full((_L,), 1, jnp.int32)
            zero = jnp.full((_L,), 0, jnp.int32)
            for g in range(groups):
                vals = vals_v[pl.ds(g * _L, _L)]
                hit = (vals > _THRESHOLD) & valids[g]
                out_v[pl.ds(g * _L, _L)] = jnp.where(hit, one, zero)
            pltpu.sync_copy(out_v, out_hbm.at[pl.ds(k * B, B)])

        do_k(wid)

        @pl.when(wid < K - _NS)
        def _():
            do_k(wid + _NS)

    run = pl.kernel(
        body,
        out_type=jax.ShapeDtypeStruct((K * B,), jnp.int32),
        mesh=mesh,
        compiler_params=pltpu.CompilerParams(needs_layout_passes=False),
        scratch_types=[
            pltpu.VMEM((2 * B,), jnp.int32),
            pltpu.VMEM((B,), jnp.int32),
            pltpu.VMEM((B,), jnp.float32),
            pltpu.VMEM((B,), jnp.int32),
            pltpu.SemaphoreType.DMA,
        ],
    )
    out = run(c_flat, hm_flat)
    return (out > 0).reshape(K, B).T


def kernel(coords, heatmaps):
    B, K, H, W = heatmaps.shape
    c_flat = coords.astype(jnp.int32).transpose(1, 2, 0).reshape(-1)
    hm_flat = heatmaps.transpose(1, 2, 3, 0).reshape(-1)
    return _run(c_flat, hm_flat, B, K, H, W)
